# baseline (speedup 1.0000x reference)
import functools

import jax
import jax.numpy as jnp
from jax import lax
from jax.experimental import pallas as pl
from jax.experimental.pallas import tpu as pltpu
from jax.experimental.pallas import tpu_sc as plsc

N = 10000
D = 128
NP = 10240
NC = 2
NS = 16
NW = NC * NS
B = 128
ROWS_PER_TILE = NP // NS
R = 256
GRID = NP // R


def _sc_mesh():
    return plsc.VectorSubcoreMesh(core_axis_name="c", subcore_axis_name="s",
                                  num_cores=NC, num_subcores=NS)


def _make_deg_kernel(epw):
    nbatch = epw // B

    @functools.partial(
        pl.kernel,
        out_type=jax.ShapeDtypeStruct((NW, NP), jnp.float32),
        mesh=_sc_mesh(),
        scratch_types=[
            pltpu.VMEM((B,), jnp.int32),
            pltpu.VMEM((NP,), jnp.float32),
        ],
        compiler_params=pltpu.CompilerParams(needs_layout_passes=False),
    )
    def deg_kernel(dst_hbm, zeros_hbm, out_hbm, dstv, degv):
        c = lax.axis_index("c")
        s = lax.axis_index("s")
        w = c * NS + s
        pltpu.sync_copy(zeros_hbm, degv)
        base = w * epw
        ones = jnp.full((16,), 1.0, jnp.float32)

        def step(j, carry):
            pltpu.sync_copy(dst_hbm.at[pl.ds(base + j * B, B)], dstv)
            for k in range(B // 16):
                idx = dstv[pl.ds(k * 16, 16)]
                plsc.addupdate_scatter(degv, [idx], ones)
            return carry

        lax.fori_loop(0, nbatch, step, 0)
        pltpu.sync_copy(degv, out_hbm.at[w])

    return deg_kernel


def _make_edge_kernel(epw):
    nbatch = epw // B

    assert nbatch % 2 == 0

    @functools.partial(
        pl.kernel,
        out_type=jax.ShapeDtypeStruct((NC * NP, D), jnp.float32),
        mesh=_sc_mesh(),
        scratch_types=[
            pltpu.VMEM((2, B), jnp.int32),
            pltpu.VMEM((2, B), jnp.int32),
            pltpu.VMEM((B, D), jnp.float32),
            pltpu.VMEM((B, D), jnp.float32),
            pltpu.VMEM_SHARED((NP, D), jnp.float32),
            pltpu.SemaphoreType.DMA,
            pltpu.SemaphoreType.DMA,
            pltpu.SemaphoreType.DMA,
            pltpu.SemaphoreType.DMA,
        ],
    )
    def edge_kernel(hs_hbm, idx_hbm, zeros_hbm, out_hbm,
                    idx0, idx1, rows0, rows1, acc,
                    sem_i0, sem_i1, sem_g0, sem_g1):
        c = lax.axis_index("c")
        s = lax.axis_index("s")
        idxb = (idx0, idx1)
        rowsb = (rows0, rows1)
        sem_i = (sem_i0, sem_i1)
        sem_g = (sem_g0, sem_g1)
        pltpu.sync_copy(zeros_hbm, rows0)
        for z in range(ROWS_PER_TILE // B):
            pltpu.sync_copy(rows0, acc.at[pl.ds(s * ROWS_PER_TILE + z * B, B)])
        plsc.subcore_barrier()
        w = c * NS + s
        base = w * nbatch
        last = base + nbatch - 1

        pltpu.sync_copy(idx_hbm.at[base], idx0)
        pltpu.async_copy(hs_hbm.at[idx0.at[0]], rows0, sem_g0)
        pltpu.async_copy(idx_hbm.at[base + 1], idx1, sem_i1)

        def step(j2, carry):
            j = base + 2 * j2
            for b in (0, 1):
                nxt = jnp.minimum(j + b + 1, last)
                nxt2 = jnp.minimum(j + b + 2, last)
                pltpu.make_async_copy(idx_hbm.at[nxt], idxb[b ^ 1],
                                      sem_i[b ^ 1]).wait()
                pltpu.async_copy(hs_hbm.at[idxb[b ^ 1].at[0]], rowsb[b ^ 1],
                                 sem_g[b ^ 1])
                pltpu.make_async_copy(hs_hbm.at[idxb[b].at[0]], rowsb[b],
                                      sem_g[b]).wait()
                pltpu.sync_copy(rowsb[b], acc.at[idxb[b].at[1]], add=True)
                pltpu.async_copy(idx_hbm.at[nxt2], idxb[b], sem_i[b])
            return carry

        lax.fori_loop(0, nbatch // 2, step, 0, unroll=False)
        pltpu.make_async_copy(idx_hbm.at[last], idx1, sem_i1).wait()
        pltpu.make_async_copy(hs_hbm.at[idx0.at[0]], rows0, sem_g0).wait()
        plsc.subcore_barrier()
        off = s * ROWS_PER_TILE
        for z in range(ROWS_PER_TILE // B):
            pltpu.sync_copy(acc.at[pl.ds(off + z * B, B)], rows0)
            pltpu.sync_copy(rows0, out_hbm.at[pl.ds(c * NP + off + z * B, B)])

    return edge_kernel


def _prep_body(x_ref, w_ref, d_ref, hs_ref, dinv_ref):
    deg = jnp.sum(d_ref[...], axis=0).reshape(R, 1) + 1.0
    rows = lax.broadcasted_iota(jnp.int32, (R, 1), 0) + pl.program_id(0) * R
    dinv = jnp.where(rows < N, lax.rsqrt(deg), 0.0)
    dinv_ref[...] = dinv
    hs_ref[...] = jnp.dot(x_ref[...], w_ref[...],
                          preferred_element_type=jnp.float32) * dinv


def _combine_body(a0_ref, a1_ref, hs_ref, dinv_ref, b_ref, w_ref, out_ref):
    dinv = dinv_ref[...]
    t = (a0_ref[...] + a1_ref[...] + hs_ref[...]) * dinv + b_ref[...]
    t = jnp.maximum(t, 0.0)
    out_ref[...] = jnp.dot(t, w_ref[...],
                           preferred_element_type=jnp.float32) * dinv


def _final_body(a0_ref, a1_ref, hs_ref, dinv_ref, b_ref, out_ref):
    t = (a0_ref[...] + a1_ref[...] + hs_ref[...]) * dinv_ref[...] + b_ref[...]
    out_ref[...] = jnp.maximum(t, 0.0)


def _rows_spec(width):
    return pl.BlockSpec((R, width), lambda i: (i, 0))


def _part_specs(width):
    return (pl.BlockSpec((R, width), lambda i: (i, 0)),
            pl.BlockSpec((R, width), lambda i: (i + GRID, 0)))


_FULL_W = pl.BlockSpec((D, D), lambda i: (0, 0))
_FULL_B = pl.BlockSpec((1, D), lambda i: (0, 0))

_prep_call = pl.pallas_call(
    _prep_body,
    grid=(GRID,),
    in_specs=[_rows_spec(D), _FULL_W, pl.BlockSpec((NW, R), lambda i: (0, i))],
    out_specs=[_rows_spec(D), _rows_spec(1)],
    out_shape=[jax.ShapeDtypeStruct((NP, D), jnp.float32),
               jax.ShapeDtypeStruct((NP, 1), jnp.float32)],
)

_combine_call = pl.pallas_call(
    _combine_body,
    grid=(GRID,),
    in_specs=[*_part_specs(D), _rows_spec(D), _rows_spec(1), _FULL_B, _FULL_W],
    out_specs=_rows_spec(D),
    out_shape=jax.ShapeDtypeStruct((NP, D), jnp.float32),
)

_final_call = pl.pallas_call(
    _final_body,
    grid=(GRID,),
    in_specs=[*_part_specs(D), _rows_spec(D), _rows_spec(1), _FULL_B],
    out_specs=_rows_spec(D),
    out_shape=jax.ShapeDtypeStruct((NP, D), jnp.float32),
)


def kernel(x, edge_index, W1, b1, W2, b2, W3, b3, W4, b4):
    E = edge_index.shape[1]
    epw = -(-E // (NW * 2 * B)) * 2 * B
    ep = epw * NW
    pad = ep - E
    src = jnp.concatenate(
        [edge_index[0], jnp.full((pad,), N, jnp.int32)]).astype(jnp.int32)
    dst = jnp.concatenate(
        [edge_index[1], jnp.full((pad,), N, jnp.int32)]).astype(jnp.int32)
    packed = jnp.stack([src.reshape(-1, B), dst.reshape(-1, B)], axis=1)
    x_p = jnp.zeros((NP, D), jnp.float32).at[:N].set(x)

    zeros_d = jnp.zeros((B, D), jnp.float32)

    deg_kernel = _make_deg_kernel(epw)
    edge_kernel = _make_edge_kernel(epw)

    degp = deg_kernel(dst, jnp.zeros((NP,), jnp.float32))
    hs, dinv = _prep_call(x_p, W1, degp)

    b1r = b1.reshape(1, D)
    b2r = b2.reshape(1, D)
    b3r = b3.reshape(1, D)
    b4r = b4.reshape(1, D)

    def agg(h):
        return edge_kernel(h, packed, zeros_d)

    accp = agg(hs)
    hs = _combine_call(accp, accp, hs, dinv, b1r, W2)

    accp = agg(hs)
    hs = _combine_call(accp, accp, hs, dinv, b2r, W3)

    accp = agg(hs)
    hs = _combine_call(accp, accp, hs, dinv, b3r, W4)

    accp = agg(hs)
    out = _final_call(accp, accp, hs, dinv, b4r)
    return out[:N]


# speedup vs baseline: 1.8105x; 1.8105x over previous
import functools

import jax
import jax.numpy as jnp
from jax import lax
from jax.experimental import pallas as pl
from jax.experimental.pallas import tpu as pltpu
from jax.experimental.pallas import tpu_sc as plsc

N = 10000
D = 128
NP = 10240
NC = 2
NS = 16
NW = NC * NS
B = 128
ROWS_PER_TILE = NP // NS
R = 256
GRID = NP // R


def _sc_mesh():
    return plsc.VectorSubcoreMesh(core_axis_name="c", subcore_axis_name="s",
                                  num_cores=NC, num_subcores=NS)


def _make_deg_kernel(epw):
    nbatch = epw // B

    @functools.partial(
        pl.kernel,
        out_type=jax.ShapeDtypeStruct((NW, NP), jnp.float32),
        mesh=_sc_mesh(),
        scratch_types=[
            pltpu.VMEM((B,), jnp.int32),
            pltpu.VMEM((NP,), jnp.float32),
        ],
        compiler_params=pltpu.CompilerParams(needs_layout_passes=False),
    )
    def deg_kernel(dst_hbm, zeros_hbm, out_hbm, dstv, degv):
        c = lax.axis_index("c")
        s = lax.axis_index("s")
        w = c * NS + s
        pltpu.sync_copy(zeros_hbm, degv)
        base = w * epw
        ones = jnp.full((16,), 1.0, jnp.float32)

        def step(j, carry):
            pltpu.sync_copy(dst_hbm.at[pl.ds(base + j * B, B)], dstv)
            for k in range(B // 16):
                idx = dstv[pl.ds(k * 16, 16)]
                plsc.addupdate_scatter(degv, [idx], ones)
            return carry

        lax.fori_loop(0, nbatch, step, 0)
        pltpu.sync_copy(degv, out_hbm.at[w])

    return deg_kernel


def _make_edge_kernel(nb0, nb1):
    assert nb0 % 2 == 0 and nb1 % 2 == 0

    @functools.partial(
        pl.kernel,
        out_type=jax.ShapeDtypeStruct((NC * NP, D), jnp.float32),
        mesh=_sc_mesh(),
        scratch_types=[
            pltpu.VMEM((2, B), jnp.int32),
            pltpu.VMEM((2, B), jnp.int32),
            pltpu.VMEM((B, D), jnp.float32),
            pltpu.VMEM((B, D), jnp.float32),
            pltpu.VMEM_SHARED((NP, D), jnp.float32),
            pltpu.SemaphoreType.DMA,
            pltpu.SemaphoreType.DMA,
            pltpu.SemaphoreType.DMA,
            pltpu.SemaphoreType.DMA,
        ],
    )
    def edge_kernel(hs_hbm, idx_hbm, zeros_hbm, out_hbm,
                    idx0, idx1, rows0, rows1, acc,
                    sem_i0, sem_i1, sem_g0, sem_g1):
        c = lax.axis_index("c")
        s = lax.axis_index("s")
        idxb = (idx0, idx1)
        rowsb = (rows0, rows1)
        sem_i = (sem_i0, sem_i1)
        sem_g = (sem_g0, sem_g1)
        pltpu.sync_copy(zeros_hbm, rows0)
        for z in range(ROWS_PER_TILE // B):
            pltpu.sync_copy(rows0, acc.at[pl.ds(s * ROWS_PER_TILE + z * B, B)])
        plsc.subcore_barrier()
        npt = jnp.where(c == 0, nb0, nb1)
        base = jnp.where(c == 0, s * nb0, NS * nb0 + s * nb1)
        last = base + npt - 1

        pltpu.sync_copy(idx_hbm.at[base], idx0)
        pltpu.async_copy(hs_hbm.at[idx0.at[0]], rows0, sem_g0)
        pltpu.async_copy(idx_hbm.at[base + 1], idx1, sem_i1)

        def step(j2, carry):
            j = base + 2 * j2
            for b in (0, 1):
                nxt = jnp.minimum(j + b + 1, last)
                nxt2 = jnp.minimum(j + b + 2, last)
                pltpu.make_async_copy(idx_hbm.at[nxt], idxb[b ^ 1],
                                      sem_i[b ^ 1]).wait()
                pltpu.async_copy(hs_hbm.at[idxb[b ^ 1].at[0]], rowsb[b ^ 1],
                                 sem_g[b ^ 1])
                pltpu.make_async_copy(hs_hbm.at[idxb[b].at[0]], rowsb[b],
                                      sem_g[b]).wait()
                pltpu.sync_copy(rowsb[b], acc.at[idxb[b].at[1]], add=True)
                pltpu.async_copy(idx_hbm.at[nxt2], idxb[b], sem_i[b])
            return carry

        lax.fori_loop(0, npt // 2, step, 0, unroll=False)
        pltpu.make_async_copy(idx_hbm.at[last], idx1, sem_i1).wait()
        pltpu.make_async_copy(hs_hbm.at[idx0.at[0]], rows0, sem_g0).wait()
        plsc.subcore_barrier()
        off = s * ROWS_PER_TILE
        for z in range(ROWS_PER_TILE // B):
            pltpu.sync_copy(acc.at[pl.ds(off + z * B, B)], rows0)
            pltpu.sync_copy(rows0, out_hbm.at[pl.ds(c * NP + off + z * B, B)])

    return edge_kernel


def _prep_body(x_ref, w_ref, d_ref, hs_ref, dinv_ref):
    deg = jnp.sum(d_ref[...], axis=0).reshape(R, 1) + 1.0
    rows = lax.broadcasted_iota(jnp.int32, (R, 1), 0) + pl.program_id(0) * R
    dinv = jnp.where(rows < N, lax.rsqrt(deg), 0.0)
    dinv_ref[...] = dinv
    hs_ref[...] = jnp.dot(x_ref[...], w_ref[...],
                          preferred_element_type=jnp.float32) * dinv


def _combine_body(a0_ref, a1_ref, hs_ref, dinv_ref, b_ref, w_ref, out_ref):
    dinv = dinv_ref[...]
    t = (a0_ref[...] + a1_ref[...] + hs_ref[...]) * dinv + b_ref[...]
    t = jnp.maximum(t, 0.0)
    out_ref[...] = jnp.dot(t, w_ref[...],
                           preferred_element_type=jnp.float32) * dinv


def _final_body(a0_ref, a1_ref, hs_ref, dinv_ref, b_ref, out_ref):
    t = (a0_ref[...] + a1_ref[...] + hs_ref[...]) * dinv_ref[...] + b_ref[...]
    out_ref[...] = jnp.maximum(t, 0.0)


def _rows_spec(width):
    return pl.BlockSpec((R, width), lambda i: (i, 0))


def _part_specs(width):
    return (pl.BlockSpec((R, width), lambda i: (i, 0)),
            pl.BlockSpec((R, width), lambda i: (i + GRID, 0)))


_FULL_W = pl.BlockSpec((D, D), lambda i: (0, 0))
_FULL_B = pl.BlockSpec((1, D), lambda i: (0, 0))

_prep_call = pl.pallas_call(
    _prep_body,
    grid=(GRID,),
    in_specs=[_rows_spec(D), _FULL_W, pl.BlockSpec((NW, R), lambda i: (0, i))],
    out_specs=[_rows_spec(D), _rows_spec(1)],
    out_shape=[jax.ShapeDtypeStruct((NP, D), jnp.float32),
               jax.ShapeDtypeStruct((NP, 1), jnp.float32)],
)

_combine_call = pl.pallas_call(
    _combine_body,
    grid=(GRID,),
    in_specs=[*_part_specs(D), _rows_spec(D), _rows_spec(1), _FULL_B, _FULL_W],
    out_specs=_rows_spec(D),
    out_shape=jax.ShapeDtypeStruct((NP, D), jnp.float32),
)

_final_call = pl.pallas_call(
    _final_body,
    grid=(GRID,),
    in_specs=[*_part_specs(D), _rows_spec(D), _rows_spec(1), _FULL_B],
    out_specs=_rows_spec(D),
    out_shape=jax.ShapeDtypeStruct((NP, D), jnp.float32),
)


def kernel(x, edge_index, W1, b1, W2, b2, W3, b3, W4, b4):
    E = edge_index.shape[1]
    SPLIT = 0.75
    nbt = -(-E // (NS * B))
    nbt += nbt % 2
    nb0 = int(nbt * SPLIT + 0.5)
    nb0 -= nb0 % 2
    nb1 = nbt - nb0
    ep = nbt * NS * B
    pad = ep - E
    src = jnp.concatenate(
        [edge_index[0], jnp.full((pad,), N, jnp.int32)]).astype(jnp.int32)
    dst = jnp.concatenate(
        [edge_index[1], jnp.full((pad,), N, jnp.int32)]).astype(jnp.int32)
    packed = jnp.stack([src.reshape(-1, B), dst.reshape(-1, B)], axis=1)
    epw = -(-E // (NW * B)) * B
    dpad = epw * NW - E
    dst_deg = jnp.concatenate(
        [edge_index[1], jnp.full((dpad,), N, jnp.int32)]).astype(jnp.int32)
    x_p = jnp.zeros((NP, D), jnp.float32).at[:N].set(x)

    zeros_d = jnp.zeros((B, D), jnp.float32)

    deg_kernel = _make_deg_kernel(epw)
    edge_kernel = _make_edge_kernel(nb0, nb1)

    degp = deg_kernel(dst_deg, jnp.zeros((NP,), jnp.float32))
    hs, dinv = _prep_call(x_p, W1, degp)

    b1r = b1.reshape(1, D)
    b2r = b2.reshape(1, D)
    b3r = b3.reshape(1, D)
    b4r = b4.reshape(1, D)

    def agg(h):
        return edge_kernel(h, packed, zeros_d)

    accp = agg(hs)
    hs = _combine_call(accp, accp, hs, dinv, b1r, W2)

    accp = agg(hs)
    hs = _combine_call(accp, accp, hs, dinv, b2r, W3)

    accp = agg(hs)
    hs = _combine_call(accp, accp, hs, dinv, b3r, W4)

    accp = agg(hs)
    out = _final_call(accp, accp, hs, dinv, b4r)
    return out[:N]


# speedup vs baseline: 1.8468x; 1.0200x over previous
import functools

import jax
import jax.numpy as jnp
from jax import lax
from jax.experimental import pallas as pl
from jax.experimental.pallas import tpu as pltpu
from jax.experimental.pallas import tpu_sc as plsc

N = 10000
D = 128
NP = 10240
NC = 2
NS = 16
NW = NC * NS
B = 128
ROWS_PER_TILE = NP // NS
R = 256
GRID = NP // R


def _sc_mesh():
    return plsc.VectorSubcoreMesh(core_axis_name="c", subcore_axis_name="s",
                                  num_cores=NC, num_subcores=NS)


def _make_deg_kernel(epw):
    nbatch = epw // B

    @functools.partial(
        pl.kernel,
        out_type=jax.ShapeDtypeStruct((NW, NP), jnp.float32),
        mesh=_sc_mesh(),
        scratch_types=[
            pltpu.VMEM((B,), jnp.int32),
            pltpu.VMEM((NP,), jnp.float32),
        ],
        compiler_params=pltpu.CompilerParams(needs_layout_passes=False),
    )
    def deg_kernel(dst_hbm, zeros_hbm, out_hbm, dstv, degv):
        c = lax.axis_index("c")
        s = lax.axis_index("s")
        w = c * NS + s
        pltpu.sync_copy(zeros_hbm, degv)
        base = w * epw
        ones = jnp.full((16,), 1.0, jnp.float32)

        def step(j, carry):
            pltpu.sync_copy(dst_hbm.at[pl.ds(base + j * B, B)], dstv)
            for k in range(B // 16):
                idx = dstv[pl.ds(k * 16, 16)]
                plsc.addupdate_scatter(degv, [idx], ones)
            return carry

        lax.fori_loop(0, nbatch, step, 0)
        pltpu.sync_copy(degv, out_hbm.at[w])

    return deg_kernel


def _make_edge_kernel(nb0, nb1):
    assert nb0 % 2 == 0 and nb1 % 2 == 0

    @functools.partial(
        pl.kernel,
        out_type=jax.ShapeDtypeStruct((NC * NP, D), jnp.float32),
        mesh=_sc_mesh(),
        scratch_types=[
            pltpu.VMEM((2, B), jnp.int32),
            pltpu.VMEM((2, B), jnp.int32),
            pltpu.VMEM((B, D), jnp.float32),
            pltpu.VMEM((B, D), jnp.float32),
            pltpu.VMEM_SHARED((NP, D), jnp.float32),
            pltpu.SemaphoreType.DMA,
            pltpu.SemaphoreType.DMA,
            pltpu.SemaphoreType.DMA,
            pltpu.SemaphoreType.DMA,
        ],
    )
    def edge_kernel(hs_hbm, idx_hbm, zeros_hbm, out_hbm,
                    idx0, idx1, rows0, rows1, acc,
                    sem_i0, sem_i1, sem_g0, sem_g1):
        c = lax.axis_index("c")
        s = lax.axis_index("s")
        idxb = (idx0, idx1)
        rowsb = (rows0, rows1)
        sem_i = (sem_i0, sem_i1)
        sem_g = (sem_g0, sem_g1)
        pltpu.sync_copy(zeros_hbm, rows0)
        for z in range(ROWS_PER_TILE // B):
            pltpu.sync_copy(rows0, acc.at[pl.ds(s * ROWS_PER_TILE + z * B, B)])
        plsc.subcore_barrier()
        npt = jnp.where(c == 0, nb0, nb1)
        base = jnp.where(c == 0, s * nb0, NS * nb0 + s * nb1)
        last = base + npt - 1

        pltpu.sync_copy(idx_hbm.at[base], idx0)
        pltpu.async_copy(hs_hbm.at[idx0.at[0]], rows0, sem_g0)
        pltpu.async_copy(idx_hbm.at[base + 1], idx1, sem_i1)

        def step(j2, carry):
            j = base + 2 * j2
            for b in (0, 1):
                nxt = jnp.minimum(j + b + 1, last)
                nxt2 = jnp.minimum(j + b + 2, last)
                pltpu.make_async_copy(idx_hbm.at[nxt], idxb[b ^ 1],
                                      sem_i[b ^ 1]).wait()
                pltpu.async_copy(hs_hbm.at[idxb[b ^ 1].at[0]], rowsb[b ^ 1],
                                 sem_g[b ^ 1])
                pltpu.make_async_copy(hs_hbm.at[idxb[b].at[0]], rowsb[b],
                                      sem_g[b]).wait()
                pltpu.sync_copy(rowsb[b], acc.at[idxb[b].at[1]], add=True)
                pltpu.async_copy(idx_hbm.at[nxt2], idxb[b], sem_i[b])
            return carry

        lax.fori_loop(0, npt // 2, step, 0, unroll=False)
        pltpu.make_async_copy(idx_hbm.at[last], idx1, sem_i1).wait()
        pltpu.make_async_copy(hs_hbm.at[idx0.at[0]], rows0, sem_g0).wait()
        plsc.subcore_barrier()
        off = s * ROWS_PER_TILE
        for z in range(ROWS_PER_TILE // B):
            pltpu.sync_copy(acc.at[pl.ds(off + z * B, B)], rows0)
            pltpu.sync_copy(rows0, out_hbm.at[pl.ds(c * NP + off + z * B, B)])

    return edge_kernel


def _prep_body(x_ref, w_ref, d_ref, hs_ref, dinv_ref):
    deg = jnp.sum(d_ref[...], axis=0).reshape(R, 1) + 1.0
    rows = lax.broadcasted_iota(jnp.int32, (R, 1), 0) + pl.program_id(0) * R
    dinv = jnp.where(rows < N, lax.rsqrt(deg), 0.0)
    dinv_ref[...] = dinv
    hs_ref[...] = jnp.dot(x_ref[...], w_ref[...],
                          preferred_element_type=jnp.float32) * dinv


def _combine_body(a0_ref, a1_ref, hs_ref, dinv_ref, b_ref, w_ref, out_ref):
    dinv = dinv_ref[...]
    t = (a0_ref[...] + a1_ref[...] + hs_ref[...]) * dinv + b_ref[...]
    t = jnp.maximum(t, 0.0)
    out_ref[...] = jnp.dot(t, w_ref[...],
                           preferred_element_type=jnp.float32) * dinv


def _final_body(a0_ref, a1_ref, hs_ref, dinv_ref, b_ref, out_ref):
    t = (a0_ref[...] + a1_ref[...] + hs_ref[...]) * dinv_ref[...] + b_ref[...]
    out_ref[...] = jnp.maximum(t, 0.0)


def _rows_spec(width):
    return pl.BlockSpec((R, width), lambda i: (i, 0))


def _part_specs(width):
    return (pl.BlockSpec((R, width), lambda i: (i, 0)),
            pl.BlockSpec((R, width), lambda i: (i + GRID, 0)))


_FULL_W = pl.BlockSpec((D, D), lambda i: (0, 0))
_FULL_B = pl.BlockSpec((1, D), lambda i: (0, 0))

_prep_call = pl.pallas_call(
    _prep_body,
    grid=(GRID,),
    in_specs=[_rows_spec(D), _FULL_W, pl.BlockSpec((NW, R), lambda i: (0, i))],
    out_specs=[_rows_spec(D), _rows_spec(1)],
    out_shape=[jax.ShapeDtypeStruct((NP, D), jnp.float32),
               jax.ShapeDtypeStruct((NP, 1), jnp.float32)],
)

_combine_call = pl.pallas_call(
    _combine_body,
    grid=(GRID,),
    in_specs=[*_part_specs(D), _rows_spec(D), _rows_spec(1), _FULL_B, _FULL_W],
    out_specs=_rows_spec(D),
    out_shape=jax.ShapeDtypeStruct((NP, D), jnp.float32),
)

_final_call = pl.pallas_call(
    _final_body,
    grid=(GRID,),
    in_specs=[*_part_specs(D), _rows_spec(D), _rows_spec(1), _FULL_B],
    out_specs=_rows_spec(D),
    out_shape=jax.ShapeDtypeStruct((NP, D), jnp.float32),
)


def kernel(x, edge_index, W1, b1, W2, b2, W3, b3, W4, b4):
    E = edge_index.shape[1]
    SPLIT = 0.785
    nbt = -(-E // (NS * B))
    nbt += nbt % 2
    nb0 = int(nbt * SPLIT + 0.5)
    nb0 -= nb0 % 2
    nb1 = nbt - nb0
    ep = nbt * NS * B
    pad = ep - E
    src = jnp.concatenate(
        [edge_index[0], jnp.full((pad,), N, jnp.int32)]).astype(jnp.int32)
    dst = jnp.concatenate(
        [edge_index[1], jnp.full((pad,), N, jnp.int32)]).astype(jnp.int32)
    packed = jnp.stack([src.reshape(-1, B), dst.reshape(-1, B)], axis=1)
    epw = -(-E // (NW * B)) * B
    dpad = epw * NW - E
    dst_deg = jnp.concatenate(
        [edge_index[1], jnp.full((dpad,), N, jnp.int32)]).astype(jnp.int32)
    x_p = jnp.zeros((NP, D), jnp.float32).at[:N].set(x)

    zeros_d = jnp.zeros((B, D), jnp.float32)

    deg_kernel = _make_deg_kernel(epw)
    edge_kernel = _make_edge_kernel(nb0, nb1)

    degp = deg_kernel(dst_deg, jnp.zeros((NP,), jnp.float32))
    hs, dinv = _prep_call(x_p, W1, degp)

    b1r = b1.reshape(1, D)
    b2r = b2.reshape(1, D)
    b3r = b3.reshape(1, D)
    b4r = b4.reshape(1, D)

    def agg(h):
        return edge_kernel(h, packed, zeros_d)

    accp = agg(hs)
    hs = _combine_call(accp, accp, hs, dinv, b1r, W2)

    accp = agg(hs)
    hs = _combine_call(accp, accp, hs, dinv, b2r, W3)

    accp = agg(hs)
    hs = _combine_call(accp, accp, hs, dinv, b3r, W4)

    accp = agg(hs)
    out = _final_call(accp, accp, hs, dinv, b4r)
    return out[:N]
